# baseline (speedup 1.0000x reference)
import functools
import math

import jax
import jax.numpy as jnp
from jax import lax
from jax.experimental import pallas as pl
from jax.experimental.pallas import tpu as pltpu
from jax.experimental.pallas import tpu_sc as plsc

T = 2048
C = 768
FF = 3072
E = 64
TT = 64
PT = T + E * TT

NC = 2
NS = 16
NW = NC * NS
TPW = T // NW

_SQRT2 = math.sqrt(2.0)


def _dispatch_body(x_ref, rw_ref, dest_ref, wt_ref, base_ref, ntl_ref):
    xx = x_ref[...]
    rw = rw_ref[...]
    logits = lax.dot_general(xx, rw, (((1,), (1,)), ((), ())),
                             preferred_element_type=jnp.float32)
    m = jnp.max(logits, axis=1, keepdims=True)
    s = jnp.sum(jnp.exp(logits - m), axis=1, keepdims=True)
    p = 1.0 / s
    wt_ref[...] = p / (p + 1e-9)

    lane_e = lax.broadcasted_iota(jnp.int32, (T, E), 1).astype(jnp.float32)
    cand = jnp.where(logits == m, lane_e, float(E))
    eid_f = jnp.min(cand, axis=1, keepdims=True)
    onehot = (lane_e == eid_f).astype(jnp.float32)

    CH = 128
    r_i = lax.broadcasted_iota(jnp.int32, (CH, CH), 0)
    c_i = lax.broadcasted_iota(jnp.int32, (CH, CH), 1)
    lt = (c_i < r_i).astype(jnp.float32)
    tot = jnp.zeros((1, E), jnp.float32)
    rank_rows = []
    for k in range(T // CH):
        oh = onehot[k * CH:(k + 1) * CH]
        rk = lax.dot_general(lt, oh, (((1,), (0,)), ((), ())),
                             preferred_element_type=jnp.float32) + tot
        rank_rows.append(rk)
        tot = tot + jnp.sum(oh, axis=0, keepdims=True)
    ranks = jnp.concatenate(rank_rows, axis=0)

    counts_i = tot.astype(jnp.int32)
    padded_i = ((counts_i + (TT - 1)) // TT) * TT
    ntl_ref[...] = padded_i // TT
    u_i = lax.broadcasted_iota(jnp.int32, (E, E), 0)
    u_j = lax.broadcasted_iota(jnp.int32, (E, E), 1)
    su = (u_i < u_j).astype(jnp.float32)
    base_f = lax.dot_general(padded_i.astype(jnp.float32), su,
                             (((1,), (0,)), ((), ())),
                             preferred_element_type=jnp.float32)
    base_ref[...] = base_f.astype(jnp.int32)
    rank_t = jnp.sum(ranks * onehot, axis=1, keepdims=True)
    base_t = jnp.sum(onehot * base_f, axis=1, keepdims=True)
    dest_ref[...] = (rank_t + base_t).astype(jnp.int32)


def _dispatch(xf, router_w, interpret=False):
    return pl.pallas_call(
        _dispatch_body,
        out_shape=[
            jax.ShapeDtypeStruct((T, 1), jnp.int32),
            jax.ShapeDtypeStruct((T, 1), jnp.float32),
            jax.ShapeDtypeStruct((1, E), jnp.int32),
            jax.ShapeDtypeStruct((1, E), jnp.int32),
        ],
        interpret=interpret,
    )(xf, router_w)


def _scatter_body(x_hbm, dest_hbm, wt_hbm, xs_hbm, ws_hbm,
                  idx_v, rows_v, wt_v, sem):
    wid = lax.axis_index("s") * NC + lax.axis_index("c")
    base = wid * TPW
    pltpu.sync_copy(dest_hbm.at[pl.ds(base, TPW)], idx_v)
    pltpu.sync_copy(x_hbm.at[pl.ds(base, TPW)], rows_v)
    pltpu.async_copy(rows_v, xs_hbm.at[idx_v], sem).wait()
    pltpu.sync_copy(wt_hbm.at[pl.ds(base, TPW)], wt_v)
    pltpu.async_copy(wt_v, ws_hbm.at[idx_v], sem).wait()


@functools.lru_cache(maxsize=None)
def _scatter_call():
    return pl.kernel(
        _scatter_body,
        out_type=[
            jax.ShapeDtypeStruct((PT, C), jnp.float32),
            jax.ShapeDtypeStruct((PT,), jnp.float32),
        ],
        mesh=plsc.VectorSubcoreMesh(core_axis_name="c", subcore_axis_name="s",
                                    num_cores=NC, num_subcores=NS),
        scratch_types=[
            pltpu.VMEM((TPW,), jnp.int32),
            pltpu.VMEM((TPW, C), jnp.float32),
            pltpu.VMEM((TPW,), jnp.float32),
            pltpu.SemaphoreType.DMA,
        ],
    )


NCH1 = 4
NCH2 = 4
FC1 = FF // NCH1
CC2 = C // NCH2


def _mlp_body(base_sref, ntl_sref, x_any, w1_any, b1_ref, w2_any, b2_ref,
              wt_ref, y_any, w1_buf, w2_buf, x_tile, y_tile,
              sem_w, sem_in, sem_out):
    e = pl.program_id(0)
    b = lax.rem(e, 2)
    nb = lax.rem(e + 1, 2)

    def w_copies(ee, bb):
        cps = []
        for c in range(NCH1):
            cps.append(pltpu.make_async_copy(
                w1_any.at[ee, pl.ds(c * FC1, FC1)],
                w1_buf.at[bb, pl.ds(c * FC1, FC1)], sem_w.at[bb]))
        for c in range(NCH2):
            cps.append(pltpu.make_async_copy(
                w2_any.at[ee, pl.ds(c * CC2, CC2)],
                w2_buf.at[bb, pl.ds(c * CC2, CC2)], sem_w.at[bb]))
        return cps

    @pl.when(e == 0)
    def _prologue():
        for cp in w_copies(0, 0):
            cp.start()

    @pl.when(e + 1 < E)
    def _prefetch():
        for cp in w_copies(e + 1, nb):
            cp.start()

    for cp in w_copies(e, b):
        cp.wait()

    rs = base_sref[e]
    nt = ntl_sref[e]
    w1v = w1_buf[b]
    w2v = w2_buf[b]

    def tile_body(j, carry):
        row = pl.multiple_of(rs + j * TT, TT)
        cp = pltpu.make_async_copy(x_any.at[pl.ds(row, TT)], x_tile, sem_in)
        cp.start()
        cp.wait()
        h = lax.dot_general(x_tile[...], w1v, (((1,), (1,)), ((), ())),
                            preferred_element_type=jnp.float32)
        h = h + b1_ref[0]
        h = h * 0.5 * (1.0 + lax.erf(h / _SQRT2))
        y = lax.dot_general(h, w2v, (((1,), (1,)), ((), ())),
                            preferred_element_type=jnp.float32)
        y = (y + b2_ref[0]) * wt_ref[pl.ds(row, TT), :]
        y_tile[...] = y
        cpo = pltpu.make_async_copy(y_tile, y_any.at[pl.ds(row, TT)], sem_out)
        cpo.start()
        cpo.wait()
        return carry

    lax.fori_loop(0, nt, tile_body, 0)


def _mlp(base_i, ntl_i, xs, w1, b1, w2, b2, ws2, interpret=False):
    grid_spec = pltpu.PrefetchScalarGridSpec(
        num_scalar_prefetch=2,
        grid=(E,),
        in_specs=[
            pl.BlockSpec(memory_space=pl.ANY),
            pl.BlockSpec(memory_space=pl.ANY),
            pl.BlockSpec((1, 1, FF), lambda e, b, n: (e, 0, 0)),
            pl.BlockSpec(memory_space=pl.ANY),
            pl.BlockSpec((1, 1, C), lambda e, b, n: (e, 0, 0)),
            pl.BlockSpec((PT, 1), lambda e, b, n: (0, 0)),
        ],
        out_specs=pl.BlockSpec(memory_space=pl.ANY),
        scratch_shapes=[
            pltpu.VMEM((2, FF, C), jnp.float32),
            pltpu.VMEM((2, C, FF), jnp.float32),
            pltpu.VMEM((TT, C), jnp.float32),
            pltpu.VMEM((TT, C), jnp.float32),
            pltpu.SemaphoreType.DMA((2,)),
            pltpu.SemaphoreType.DMA,
            pltpu.SemaphoreType.DMA,
        ],
    )
    return pl.pallas_call(
        _mlp_body,
        grid_spec=grid_spec,
        out_shape=jax.ShapeDtypeStruct((PT, C), jnp.float32),
        interpret=interpret,
    )(base_i, ntl_i, xs, w1, b1.reshape(E, 1, FF), w2,
      b2.reshape(E, 1, C), ws2)


def _gather_body(ys_hbm, dest_hbm, out_hbm, idx_v, rows_v, sem):
    wid = lax.axis_index("s") * NC + lax.axis_index("c")
    base = wid * TPW
    pltpu.sync_copy(dest_hbm.at[pl.ds(base, TPW)], idx_v)
    pltpu.async_copy(ys_hbm.at[idx_v], rows_v, sem).wait()
    pltpu.sync_copy(rows_v, out_hbm.at[pl.ds(base, TPW)])


@functools.lru_cache(maxsize=None)
def _gather_call():
    return pl.kernel(
        _gather_body,
        out_type=jax.ShapeDtypeStruct((T, C), jnp.float32),
        mesh=plsc.VectorSubcoreMesh(core_axis_name="c", subcore_axis_name="s",
                                    num_cores=NC, num_subcores=NS),
        scratch_types=[
            pltpu.VMEM((TPW,), jnp.int32),
            pltpu.VMEM((TPW, C), jnp.float32),
            pltpu.SemaphoreType.DMA,
        ],
    )


def kernel(x, router_w, w1, b1, w2, b2):
    Bn, Nn, Cn = x.shape
    xf = x.reshape(T, C)
    dest2, wt2, base2, ntl2 = _dispatch(xf, router_w)
    dest = dest2.reshape(T)
    wt = wt2.reshape(T)
    xs, ws = _scatter_call()(xf, dest, wt)
    ys = _mlp(base2.reshape(E), ntl2.reshape(E), xs, w1, b1, w2, b2,
              ws.reshape(PT, 1))
    out = _gather_call()(ys, dest)
    return out.reshape(Bn, Nn, Cn)


# speedup vs baseline: 1.0022x; 1.0022x over previous
import functools
import math

import jax
import jax.numpy as jnp
from jax import lax
from jax.experimental import pallas as pl
from jax.experimental.pallas import tpu as pltpu
from jax.experimental.pallas import tpu_sc as plsc

T = 2048
C = 768
FF = 3072
E = 64
TT = 64
PT = T + E * TT

NC = 2
NS = 16
NW = NC * NS
TPW = T // NW

_SQRT2 = math.sqrt(2.0)


def _dispatch_body(x_ref, rw_ref, dest_ref, wt_ref, base_ref, ntl_ref):
    xx = x_ref[...]
    rw = rw_ref[...]
    logits = lax.dot_general(xx, rw, (((1,), (1,)), ((), ())),
                             preferred_element_type=jnp.float32)
    m = jnp.max(logits, axis=1, keepdims=True)
    s = jnp.sum(jnp.exp(logits - m), axis=1, keepdims=True)
    p = 1.0 / s
    wt_ref[...] = p / (p + 1e-9)

    lane_e = lax.broadcasted_iota(jnp.int32, (T, E), 1).astype(jnp.float32)
    cand = jnp.where(logits == m, lane_e, float(E))
    eid_f = jnp.min(cand, axis=1, keepdims=True)
    onehot = (lane_e == eid_f).astype(jnp.float32)

    CH = 128
    r_i = lax.broadcasted_iota(jnp.int32, (CH, CH), 0)
    c_i = lax.broadcasted_iota(jnp.int32, (CH, CH), 1)
    lt = (c_i < r_i).astype(jnp.float32)
    tot = jnp.zeros((1, E), jnp.float32)
    rank_rows = []
    for k in range(T // CH):
        oh = onehot[k * CH:(k + 1) * CH]
        rk = lax.dot_general(lt, oh, (((1,), (0,)), ((), ())),
                             preferred_element_type=jnp.float32) + tot
        rank_rows.append(rk)
        tot = tot + jnp.sum(oh, axis=0, keepdims=True)
    ranks = jnp.concatenate(rank_rows, axis=0)

    counts_i = tot.astype(jnp.int32)
    padded_i = ((counts_i + (TT - 1)) // TT) * TT
    ntl_ref[...] = padded_i // TT
    u_i = lax.broadcasted_iota(jnp.int32, (E, E), 0)
    u_j = lax.broadcasted_iota(jnp.int32, (E, E), 1)
    su = (u_i < u_j).astype(jnp.float32)
    base_f = lax.dot_general(padded_i.astype(jnp.float32), su,
                             (((1,), (0,)), ((), ())),
                             preferred_element_type=jnp.float32)
    base_ref[...] = base_f.astype(jnp.int32)
    rank_t = jnp.sum(ranks * onehot, axis=1, keepdims=True)
    base_t = jnp.sum(onehot * base_f, axis=1, keepdims=True)
    dest_ref[...] = (rank_t + base_t).astype(jnp.int32)


def _dispatch(xf, router_w, interpret=False):
    return pl.pallas_call(
        _dispatch_body,
        out_shape=[
            jax.ShapeDtypeStruct((T, 1), jnp.int32),
            jax.ShapeDtypeStruct((T, 1), jnp.float32),
            jax.ShapeDtypeStruct((1, E), jnp.int32),
            jax.ShapeDtypeStruct((1, E), jnp.int32),
        ],
        interpret=interpret,
    )(xf, router_w)


def _scatter_body(x_hbm, dest_hbm, wt_hbm, xs_hbm, ws_hbm,
                  idx_v, rows_v, wt_v, sem_i, sem_r, sem_w):
    wid = lax.axis_index("s") * NC + lax.axis_index("c")
    base = wid * TPW
    cp_i = pltpu.async_copy(dest_hbm.at[pl.ds(base, TPW)], idx_v, sem_i)
    cp_r = pltpu.async_copy(x_hbm.at[pl.ds(base, TPW)], rows_v, sem_r)
    cp_w = pltpu.async_copy(wt_hbm.at[pl.ds(base, TPW)], wt_v, sem_w)
    cp_i.wait()
    cp_r.wait()
    sc_r = pltpu.async_copy(rows_v, xs_hbm.at[idx_v], sem_r)
    cp_w.wait()
    sc_w = pltpu.async_copy(wt_v, ws_hbm.at[idx_v], sem_w)
    sc_r.wait()
    sc_w.wait()


@functools.lru_cache(maxsize=None)
def _scatter_call():
    return pl.kernel(
        _scatter_body,
        out_type=[
            jax.ShapeDtypeStruct((PT, C), jnp.float32),
            jax.ShapeDtypeStruct((PT,), jnp.float32),
        ],
        mesh=plsc.VectorSubcoreMesh(core_axis_name="c", subcore_axis_name="s",
                                    num_cores=NC, num_subcores=NS),
        scratch_types=[
            pltpu.VMEM((TPW,), jnp.int32),
            pltpu.VMEM((TPW, C), jnp.float32),
            pltpu.VMEM((TPW,), jnp.float32),
            pltpu.SemaphoreType.DMA,
            pltpu.SemaphoreType.DMA,
            pltpu.SemaphoreType.DMA,
        ],
    )


NCH1 = 4
NCH2 = 4
FC1 = FF // NCH1
CC2 = C // NCH2


def _mlp_body(base_sref, ntl_sref, x_any, w1_any, b1_ref, w2_any, b2_ref,
              wt_ref, y_any, w1_buf, w2_buf, x_tile, y_tile,
              sem_w, sem_in, sem_out):
    e = pl.program_id(0)
    b = lax.rem(e, 2)
    nb = lax.rem(e + 1, 2)

    def w_copies(ee, bb):
        cps = []
        for c in range(NCH1):
            cps.append(pltpu.make_async_copy(
                w1_any.at[ee, pl.ds(c * FC1, FC1)],
                w1_buf.at[bb, pl.ds(c * FC1, FC1)], sem_w.at[bb]))
        for c in range(NCH2):
            cps.append(pltpu.make_async_copy(
                w2_any.at[ee, pl.ds(c * CC2, CC2)],
                w2_buf.at[bb, pl.ds(c * CC2, CC2)], sem_w.at[bb]))
        return cps

    @pl.when(e == 0)
    def _prologue():
        for cp in w_copies(0, 0):
            cp.start()

    @pl.when(e + 1 < E)
    def _prefetch():
        for cp in w_copies(e + 1, nb):
            cp.start()

    for cp in w_copies(e, b):
        cp.wait()

    rs = base_sref[e]
    nt = ntl_sref[e]
    w1v = w1_buf[b]
    w2v = w2_buf[b]

    def tile_body(j, carry):
        row = pl.multiple_of(rs + j * TT, TT)
        cp = pltpu.make_async_copy(x_any.at[pl.ds(row, TT)], x_tile, sem_in)
        cp.start()
        cp.wait()
        h = lax.dot_general(x_tile[...], w1v, (((1,), (1,)), ((), ())),
                            preferred_element_type=jnp.float32)
        h = h + b1_ref[0]
        h = h * 0.5 * (1.0 + lax.erf(h / _SQRT2))
        y = lax.dot_general(h, w2v, (((1,), (1,)), ((), ())),
                            preferred_element_type=jnp.float32)
        y = (y + b2_ref[0]) * wt_ref[pl.ds(row, TT), :]
        y_tile[...] = y
        cpo = pltpu.make_async_copy(y_tile, y_any.at[pl.ds(row, TT)], sem_out)
        cpo.start()
        cpo.wait()
        return carry

    lax.fori_loop(0, nt, tile_body, 0)


def _mlp(base_i, ntl_i, xs, w1, b1, w2, b2, ws2, interpret=False):
    grid_spec = pltpu.PrefetchScalarGridSpec(
        num_scalar_prefetch=2,
        grid=(E,),
        in_specs=[
            pl.BlockSpec(memory_space=pl.ANY),
            pl.BlockSpec(memory_space=pl.ANY),
            pl.BlockSpec((1, 1, FF), lambda e, b, n: (e, 0, 0)),
            pl.BlockSpec(memory_space=pl.ANY),
            pl.BlockSpec((1, 1, C), lambda e, b, n: (e, 0, 0)),
            pl.BlockSpec((PT, 1), lambda e, b, n: (0, 0)),
        ],
        out_specs=pl.BlockSpec(memory_space=pl.ANY),
        scratch_shapes=[
            pltpu.VMEM((2, FF, C), jnp.float32),
            pltpu.VMEM((2, C, FF), jnp.float32),
            pltpu.VMEM((TT, C), jnp.float32),
            pltpu.VMEM((TT, C), jnp.float32),
            pltpu.SemaphoreType.DMA((2,)),
            pltpu.SemaphoreType.DMA,
            pltpu.SemaphoreType.DMA,
        ],
    )
    return pl.pallas_call(
        _mlp_body,
        grid_spec=grid_spec,
        out_shape=jax.ShapeDtypeStruct((PT, C), jnp.float32),
        interpret=interpret,
    )(base_i, ntl_i, xs, w1, b1.reshape(E, 1, FF), w2,
      b2.reshape(E, 1, C), ws2)


def _gather_body(ys_hbm, dest_hbm, out_hbm, idx_v, rows_v, sem):
    wid = lax.axis_index("s") * NC + lax.axis_index("c")
    base = wid * TPW
    pltpu.sync_copy(dest_hbm.at[pl.ds(base, TPW)], idx_v)
    pltpu.async_copy(ys_hbm.at[idx_v], rows_v, sem).wait()
    pltpu.sync_copy(rows_v, out_hbm.at[pl.ds(base, TPW)])


@functools.lru_cache(maxsize=None)
def _gather_call():
    return pl.kernel(
        _gather_body,
        out_type=jax.ShapeDtypeStruct((T, C), jnp.float32),
        mesh=plsc.VectorSubcoreMesh(core_axis_name="c", subcore_axis_name="s",
                                    num_cores=NC, num_subcores=NS),
        scratch_types=[
            pltpu.VMEM((TPW,), jnp.int32),
            pltpu.VMEM((TPW, C), jnp.float32),
            pltpu.SemaphoreType.DMA,
        ],
    )


def kernel(x, router_w, w1, b1, w2, b2):
    Bn, Nn, Cn = x.shape
    xf = x.reshape(T, C)
    dest2, wt2, base2, ntl2 = _dispatch(xf, router_w)
    dest = dest2.reshape(T)
    wt = wt2.reshape(T)
    xs, ws = _scatter_call()(xf, dest, wt)
    ys = _mlp(base2.reshape(E), ntl2.reshape(E), xs, w1, b1, w2, b2,
              ws.reshape(PT, 1))
    out = _gather_call()(ys, dest)
    return out.reshape(Bn, Nn, Cn)


# speedup vs baseline: 1.0363x; 1.0340x over previous
import functools
import math

import jax
import jax.numpy as jnp
from jax import lax
from jax.experimental import pallas as pl
from jax.experimental.pallas import tpu as pltpu
from jax.experimental.pallas import tpu_sc as plsc

T = 2048
C = 768
FF = 3072
E = 64
TT = 64
PT = T + E * TT

NC = 2
NS = 16
NW = NC * NS
TPW = T // NW

_SQRT2 = math.sqrt(2.0)


def _dispatch_body(x_ref, rw_ref, dest_ref, base_ref, ntl_ref):
    xx = x_ref[...]
    rw = rw_ref[...]
    logits = lax.dot_general(xx, rw, (((1,), (1,)), ((), ())),
                             preferred_element_type=jnp.float32)
    m = jnp.max(logits, axis=1, keepdims=True)
    lane_e = lax.broadcasted_iota(jnp.int32, (T, E), 1).astype(jnp.float32)
    cand = jnp.where(logits == m, lane_e, float(E))
    eid_f = jnp.min(cand, axis=1, keepdims=True)
    onehot = (lane_e == eid_f).astype(jnp.float32)

    CH = 128
    r_i = lax.broadcasted_iota(jnp.int32, (CH, CH), 0)
    c_i = lax.broadcasted_iota(jnp.int32, (CH, CH), 1)
    lt = (c_i < r_i).astype(jnp.float32)
    tot = jnp.zeros((1, E), jnp.float32)
    rank_rows = []
    for k in range(T // CH):
        oh = onehot[k * CH:(k + 1) * CH]
        rk = lax.dot_general(lt, oh, (((1,), (0,)), ((), ())),
                             preferred_element_type=jnp.float32) + tot
        rank_rows.append(rk)
        tot = tot + jnp.sum(oh, axis=0, keepdims=True)
    ranks = jnp.concatenate(rank_rows, axis=0)

    counts_i = tot.astype(jnp.int32)
    padded_i = ((counts_i + (TT - 1)) // TT) * TT
    ntl_ref[...] = padded_i // TT
    u_i = lax.broadcasted_iota(jnp.int32, (E, E), 0)
    u_j = lax.broadcasted_iota(jnp.int32, (E, E), 1)
    su = (u_i < u_j).astype(jnp.float32)
    base_f = lax.dot_general(padded_i.astype(jnp.float32), su,
                             (((1,), (0,)), ((), ())),
                             preferred_element_type=jnp.float32)
    base_ref[...] = base_f.astype(jnp.int32)
    rank_t = jnp.sum(ranks * onehot, axis=1, keepdims=True)
    base_t = jnp.sum(onehot * base_f, axis=1, keepdims=True)
    dest_ref[...] = (rank_t + base_t).astype(jnp.int32)


def _dispatch(xf, router_w, interpret=False):
    return pl.pallas_call(
        _dispatch_body,
        out_shape=[
            jax.ShapeDtypeStruct((T, 1), jnp.int32),
            jax.ShapeDtypeStruct((1, E), jnp.int32),
            jax.ShapeDtypeStruct((1, E), jnp.int32),
        ],
        interpret=interpret,
    )(xf, router_w)


def _scatter_body(x_hbm, dest_hbm, xs_hbm, idx_v, rows_v, sem_i, sem_r):
    wid = lax.axis_index("s") * NC + lax.axis_index("c")
    base = wid * TPW
    cp_i = pltpu.async_copy(dest_hbm.at[pl.ds(base, TPW)], idx_v, sem_i)
    cp_r = pltpu.async_copy(x_hbm.at[pl.ds(base, TPW)], rows_v, sem_r)
    cp_i.wait()
    cp_r.wait()
    pltpu.async_copy(rows_v, xs_hbm.at[idx_v], sem_r).wait()


@functools.lru_cache(maxsize=None)
def _scatter_call():
    return pl.kernel(
        _scatter_body,
        out_type=jax.ShapeDtypeStruct((PT, C), jnp.float32),
        mesh=plsc.VectorSubcoreMesh(core_axis_name="c", subcore_axis_name="s",
                                    num_cores=NC, num_subcores=NS),
        scratch_types=[
            pltpu.VMEM((TPW,), jnp.int32),
            pltpu.VMEM((TPW, C), jnp.float32),
            pltpu.SemaphoreType.DMA,
            pltpu.SemaphoreType.DMA,
        ],
    )


NCH1 = 4
NCH2 = 4
FC1 = FF // NCH1
CC2 = C // NCH2


def _mlp_body(base_sref, ntl_sref, x_any, w1_any, b1_ref, w2_any, b2_ref,
              y_any, w1_buf, w2_buf, x_tile, y_tile,
              sem_w, sem_in, sem_out):
    e = pl.program_id(0)
    b = lax.rem(e, 2)
    nb = lax.rem(e + 1, 2)

    def w_copies(ee, bb):
        cps = []
        for c in range(NCH1):
            cps.append(pltpu.make_async_copy(
                w1_any.at[ee, pl.ds(c * FC1, FC1)],
                w1_buf.at[bb, pl.ds(c * FC1, FC1)], sem_w.at[bb]))
        for c in range(NCH2):
            cps.append(pltpu.make_async_copy(
                w2_any.at[ee, pl.ds(c * CC2, CC2)],
                w2_buf.at[bb, pl.ds(c * CC2, CC2)], sem_w.at[bb]))
        return cps

    @pl.when(e == 0)
    def _prologue():
        for cp in w_copies(0, 0):
            cp.start()

    @pl.when(e + 1 < E)
    def _prefetch():
        for cp in w_copies(e + 1, nb):
            cp.start()

    for cp in w_copies(e, b):
        cp.wait()

    rs = base_sref[e]
    nt = ntl_sref[e]
    w1v = w1_buf[b]
    w2v = w2_buf[b]

    def tile_body(j, carry):
        row = pl.multiple_of(rs + j * TT, TT)
        cp = pltpu.make_async_copy(x_any.at[pl.ds(row, TT)], x_tile, sem_in)
        cp.start()
        cp.wait()
        h = lax.dot_general(x_tile[...], w1v, (((1,), (1,)), ((), ())),
                            preferred_element_type=jnp.float32)
        h = h + b1_ref[0]
        h = h * 0.5 * (1.0 + lax.erf(h / _SQRT2))
        y = lax.dot_general(h, w2v, (((1,), (1,)), ((), ())),
                            preferred_element_type=jnp.float32)
        y = y + b2_ref[0]
        y_tile[...] = y
        cpo = pltpu.make_async_copy(y_tile, y_any.at[pl.ds(row, TT)], sem_out)
        cpo.start()
        cpo.wait()
        return carry

    lax.fori_loop(0, nt, tile_body, 0)


def _mlp(base_i, ntl_i, xs, w1, b1, w2, b2, interpret=False):
    grid_spec = pltpu.PrefetchScalarGridSpec(
        num_scalar_prefetch=2,
        grid=(E,),
        in_specs=[
            pl.BlockSpec(memory_space=pl.ANY),
            pl.BlockSpec(memory_space=pl.ANY),
            pl.BlockSpec((1, 1, FF), lambda e, b, n: (e, 0, 0)),
            pl.BlockSpec(memory_space=pl.ANY),
            pl.BlockSpec((1, 1, C), lambda e, b, n: (e, 0, 0)),
        ],
        out_specs=pl.BlockSpec(memory_space=pl.ANY),
        scratch_shapes=[
            pltpu.VMEM((2, FF, C), jnp.float32),
            pltpu.VMEM((2, C, FF), jnp.float32),
            pltpu.VMEM((TT, C), jnp.float32),
            pltpu.VMEM((TT, C), jnp.float32),
            pltpu.SemaphoreType.DMA((2,)),
            pltpu.SemaphoreType.DMA,
            pltpu.SemaphoreType.DMA,
        ],
    )
    return pl.pallas_call(
        _mlp_body,
        grid_spec=grid_spec,
        out_shape=jax.ShapeDtypeStruct((PT, C), jnp.float32),
        interpret=interpret,
    )(base_i, ntl_i, xs, w1, b1.reshape(E, 1, FF), w2,
      b2.reshape(E, 1, C))


def _gather_body(ys_hbm, dest_hbm, out_hbm, idx_v, rows_v, sem):
    wid = lax.axis_index("s") * NC + lax.axis_index("c")
    base = wid * TPW
    pltpu.sync_copy(dest_hbm.at[pl.ds(base, TPW)], idx_v)
    pltpu.async_copy(ys_hbm.at[idx_v], rows_v, sem).wait()
    pltpu.sync_copy(rows_v, out_hbm.at[pl.ds(base, TPW)])


@functools.lru_cache(maxsize=None)
def _gather_call():
    return pl.kernel(
        _gather_body,
        out_type=jax.ShapeDtypeStruct((T, C), jnp.float32),
        mesh=plsc.VectorSubcoreMesh(core_axis_name="c", subcore_axis_name="s",
                                    num_cores=NC, num_subcores=NS),
        scratch_types=[
            pltpu.VMEM((TPW,), jnp.int32),
            pltpu.VMEM((TPW, C), jnp.float32),
            pltpu.SemaphoreType.DMA,
        ],
    )


def kernel(x, router_w, w1, b1, w2, b2):
    Bn, Nn, Cn = x.shape
    xf = x.reshape(T, C)
    dest2, base2, ntl2 = _dispatch(xf, router_w)
    dest = dest2.reshape(T)
    xs = _scatter_call()(xf, dest)
    ys = _mlp(base2.reshape(E), ntl2.reshape(E), xs, w1, b1, w2, b2)
    out = _gather_call()(ys, dest)
    return out.reshape(Bn, Nn, Cn)
